# baseline (speedup 1.0000x reference)
import functools

import jax
import jax.numpy as jnp
from jax import lax
from jax.experimental import pallas as pl
from jax.experimental.pallas import tpu as pltpu
from jax.experimental.pallas import tpu_sc as plsc

D = 4
L = 16
B = 8
P = 512 * 512
NC, NS, LANES = 2, 16, 16
NW = NC * NS
PPW = P // 4
CH = 8192
NCHUNK = PPW // CH
UNROLL = 8
UNROLL2 = 12

_MESH = plsc.VectorSubcoreMesh(core_axis_name="c", subcore_axis_name="s")
_SC_PARAMS = pltpu.CompilerParams(needs_layout_passes=False,
                                  use_tc_tiling_on_sc=True)
ROWS = CH // 512


def _log16(x):
    xi = plsc.bitcast(x, jnp.int32)
    e = (xi >> 23) - 127
    mf = plsc.bitcast((xi & 0x007FFFFF) | 0x3F800000, jnp.float32)
    adj = mf > 1.41421356
    mf = jnp.where(adj, mf * 0.5, mf)
    e = jnp.where(adj, e + 1, e).astype(jnp.float32)
    f = mf - 1.0
    y = jnp.full((LANES,), -0.145785363777915, jnp.float32)
    for c in (0.21555846203071044, -0.25229384063778415,
              0.33299270095527667, -0.49997879685373253):
        y = y * f + c
    return f + (f * f) * y + e * 0.6931471805599453


def _sqrt16(x):
    yi = 0x5F3759DF - (plsc.bitcast(x, jnp.int32) >> 1)
    y = plsc.bitcast(yi, jnp.float32)
    xh = 0.5 * x
    for _ in range(2):
        y = y * (1.5 - xh * y * y)
    return x * y


def _worker_id():
    return lax.axis_index("s") * NC + lax.axis_index("c")


def _issue(hbm_refs, bufs, sem, wid, img, quarter, c):
    row0 = quarter * (PPW // 512) + c * ROWS
    descs = []
    emb_h = hbm_refs[0]
    ebuf = bufs[0]
    for d in range(D):
        descs.append(pltpu.async_copy(
            emb_h.at[img, d, pl.ds(row0, ROWS), :], ebuf.at[d], sem))
    for h, b in zip(hbm_refs[1:], bufs[1:]):
        descs.append(pltpu.async_copy(h.at[img, pl.ds(row0, ROWS), :], b, sem))
    return descs


def _pass1_body(emb_h, inst_h, kn_h, tm_h, out_h,
                ebuf0, ibuf0, kbuf0, tbuf0, ebuf1, ibuf1, kbuf1, tbuf1,
                acc, sem):
    wid = _worker_id()
    img = wid // 4
    quarter = wid - img * 4
    zeros = jnp.zeros((LANES,), jnp.float32)
    for j in range(6 * L * LANES // LANES):
        acc[pl.ds(j * LANES, LANES)] = zeros
    lane = lax.iota(jnp.int32, LANES)
    ones = jnp.ones((LANES,), jnp.float32)
    hbm = (emb_h, inst_h, kn_h, tm_h)
    slots = ((ebuf0, ibuf0, kbuf0, tbuf0), (ebuf1, ibuf1, kbuf1, tbuf1))
    pend = _issue(hbm, slots[0], sem, wid, img, quarter, 0)
    for c in range(NCHUNK):
        ebuf, ibuf, kbuf, tbuf = slots[c % 2]
        for dsc in pend:
            dsc.wait()
        if c + 1 < NCHUNK:
            pend = _issue(hbm, slots[(c + 1) % 2], sem, wid, img, quarter, c + 1)

        @plsc.parallel_loop(0, CH // LANES, unroll=UNROLL)
        def _(i):
            r = i >> 5
            cc = (i & 31) * LANES
            iv = ibuf[r, pl.ds(cc, LANES)]
            kv = kbuf[r, pl.ds(cc, LANES)]
            tv = tbuf[r, pl.ds(cc, LANES)]
            ie = jnp.where(tv > 0.5, iv, 0)
            ik = jnp.where(kv > 0.5, ie, 0)
            idxk = ik * LANES + lane
            for d in range(D):
                ev = ebuf[d, r, pl.ds(cc, LANES)]
                plsc.addupdate_scatter(acc, [idxk + d * 256], ev)
            plsc.addupdate_scatter(acc, [idxk + 4 * 256], ones)
            plsc.addupdate_scatter(acc, [ie * LANES + lane + 5 * 256], ones)
    pltpu.sync_copy(acc, out_h.at[pl.ds(wid * 1536, 1536)])


def _pass2_body(emb_h, inst_h, tm_h, sums_h, out_h,
                ebuf0, ibuf0, tbuf0, ebuf1, ibuf1, tbuf1,
                sbuf, m0, m1, m2, m3, vacc, sem):
    wid = _worker_id()
    img = wid // 4
    quarter = wid - img * 4
    lane = lax.iota(jnp.int32, LANES)
    hbm = (emb_h, inst_h, tm_h)
    slots = ((ebuf0, ibuf0, tbuf0), (ebuf1, ibuf1, tbuf1))
    pend = _issue(hbm, slots[0], sem, wid, img, quarter, 0)
    pltpu.sync_copy(sums_h.at[pl.ds(img * 96, 96)], sbuf)
    cmax = jnp.maximum(sbuf[pl.ds(4 * LANES, LANES)], 1.0)
    mrefs = (m0, m1, m2, m3)
    for d in range(D):
        md = sbuf[pl.ds(d * LANES, LANES)] / cmax
        md = jnp.where(lane == 0, 0.0, md)
        mrefs[d][...] = md
    zeros = jnp.zeros((LANES,), jnp.float32)
    for j in range(L):
        vacc[pl.ds(j * LANES, LANES)] = zeros
    for c in range(NCHUNK):
        ebuf, ibuf, tbuf = slots[c % 2]
        for dsc in pend:
            dsc.wait()
        if c + 1 < NCHUNK:
            pend = _issue(hbm, slots[(c + 1) % 2], sem, wid, img, quarter, c + 1)

        @plsc.parallel_loop(0, CH // LANES, unroll=UNROLL2)
        def _(i):
            r = i >> 5
            cc = (i & 31) * LANES
            iv = ibuf[r, pl.ds(cc, LANES)]
            tv = tbuf[r, pl.ds(cc, LANES)]
            ie = jnp.where(tv > 0.5, iv, 0)
            d0 = ebuf[0, r, pl.ds(cc, LANES)] - plsc.load_gather(m0, [ie])
            d1 = ebuf[1, r, pl.ds(cc, LANES)] - plsc.load_gather(m1, [ie])
            d2 = ebuf[2, r, pl.ds(cc, LANES)] - plsc.load_gather(m2, [ie])
            d3 = ebuf[3, r, pl.ds(cc, LANES)] - plsc.load_gather(m3, [ie])
            q = d0 * d0 + d1 * d1 + d2 * d2 + d3 * d3 + 1e-12
            t = jnp.maximum(_sqrt16(q) - 0.5, 0.0)
            v = _log16(t * t + 1.0)
            plsc.addupdate_scatter(vacc, [ie * LANES + lane], v)
    pltpu.sync_copy(vacc, out_h.at[pl.ds(wid * 256, 256)])


_pass1 = pl.kernel(
    _pass1_body,
    out_type=jax.ShapeDtypeStruct((NW * 6 * 256,), jnp.float32),
    mesh=_MESH,
    compiler_params=_SC_PARAMS,
    scratch_types=[
        pltpu.VMEM((D, ROWS, 512), jnp.float32),
        pltpu.VMEM((ROWS, 512), jnp.int32),
        pltpu.VMEM((ROWS, 512), jnp.float32),
        pltpu.VMEM((ROWS, 512), jnp.float32),
        pltpu.VMEM((D, ROWS, 512), jnp.float32),
        pltpu.VMEM((ROWS, 512), jnp.int32),
        pltpu.VMEM((ROWS, 512), jnp.float32),
        pltpu.VMEM((ROWS, 512), jnp.float32),
        pltpu.VMEM((6 * 256,), jnp.float32),
        pltpu.SemaphoreType.DMA,
    ],
)

_pass2 = pl.kernel(
    _pass2_body,
    out_type=jax.ShapeDtypeStruct((NW * 256,), jnp.float32),
    mesh=_MESH,
    compiler_params=_SC_PARAMS,
    scratch_types=[
        pltpu.VMEM((D, ROWS, 512), jnp.float32),
        pltpu.VMEM((ROWS, 512), jnp.int32),
        pltpu.VMEM((ROWS, 512), jnp.float32),
        pltpu.VMEM((D, ROWS, 512), jnp.float32),
        pltpu.VMEM((ROWS, 512), jnp.int32),
        pltpu.VMEM((ROWS, 512), jnp.float32),
        pltpu.VMEM((96,), jnp.float32),
        pltpu.VMEM((LANES,), jnp.float32),
        pltpu.VMEM((LANES,), jnp.float32),
        pltpu.VMEM((LANES,), jnp.float32),
        pltpu.VMEM((LANES,), jnp.float32),
        pltpu.VMEM((256,), jnp.float32),
        pltpu.SemaphoreType.DMA,
    ],
)


def _final_body(s_ref, v_ref, o_ref):
    s = s_ref[...]
    vs = v_ref[...]
    r = lax.broadcasted_iota(jnp.int32, (L, L), 0)
    c = lax.broadcasted_iota(jnp.int32, (L, L), 1)
    eyef = (r == c).astype(jnp.float32)
    excl = (r == c) | (r == 0) | (c == 0)
    lane2 = lax.broadcasted_iota(jnp.int32, (1, L), 1)
    total = jnp.float32(0.0)
    for b in range(B):
        cnt = jnp.maximum(s[b * 6 + 4:b * 6 + 5, :], 1.0)
        ci = jnp.maximum(s[b * 6 + 5:b * 6 + 6, :], 1.0)
        mean = s[b * 6:b * 6 + 4, :] / cnt
        mean = jnp.where(lane2 == 0, 0.0, mean)
        la = vs[b:b + 1, :] / ci
        l_agg = jnp.sum(jnp.where(lane2 == 0, 0.0, la)) / 15.0
        cross = lax.dot_general(mean, mean, (((0,), (0,)), ((), ())),
                                preferred_element_type=jnp.float32)
        m2row = jnp.sum(cross * eyef, axis=0, keepdims=True)
        m2col = jnp.sum(cross * eyef, axis=1, keepdims=True)
        pd2 = jnp.maximum(m2col + m2row - 2.0 * cross, 0.0)
        pd = jnp.sqrt(pd2 + 1e-12)
        dval = jnp.log(jnp.maximum(3.0 - pd, 0.0) ** 2 + 1.0)
        l_dis = jnp.sum(jnp.where(excl, 0.0, dval)) / 210.0
        l_reg = jnp.sum(jnp.log(jnp.sqrt(m2row + 1e-12) + 1.0)) / 16.0 * 0.001
        total = total + l_agg + l_dis + l_reg
    o_ref[0, 0] = total / B


_final = pl.pallas_call(
    _final_body,
    out_shape=jax.ShapeDtypeStruct((1, 1), jnp.float32),
    out_specs=pl.BlockSpec(memory_space=pltpu.SMEM),
)


@jax.jit
def kernel(emb, instance, kernel, training_mask):
    inst = instance.astype(jnp.int32)
    p1 = _pass1(emb, inst, kernel, training_mask)
    sums6 = p1.reshape(B, 4, 6, L, LANES).sum(axis=(1, 4))
    p2 = _pass2(emb, inst, training_mask, sums6.reshape(-1))
    valsum = p2.reshape(B, 4, L, LANES).sum(axis=(1, 3))
    out = _final(sums6.reshape(B * 6, L), valsum)
    return out[0, 0]


# speedup vs baseline: 1.0892x; 1.0892x over previous
import jax
import jax.numpy as jnp
from jax import lax
from jax.experimental import pallas as pl
from jax.experimental.pallas import tpu as pltpu
from jax.experimental.pallas import tpu_sc as plsc

D = 4
L = 16
B = 8
P = 512 * 512
NC, NS, LANES = 2, 16, 16
NW = NC * NS
PPW = P // 4
CH = 8192
NCHUNK = PPW // CH
UNROLL = 8

_MESH = plsc.VectorSubcoreMesh(core_axis_name="c", subcore_axis_name="s")
_SC_PARAMS = pltpu.CompilerParams(needs_layout_passes=False,
                                  use_tc_tiling_on_sc=True)
ROWS = CH // 512


def _log16(x):
    xi = plsc.bitcast(x, jnp.int32)
    e = (xi >> 23) - 127
    mf = plsc.bitcast((xi & 0x007FFFFF) | 0x3F800000, jnp.float32)
    adj = mf > 1.41421356
    mf = jnp.where(adj, mf * 0.5, mf)
    e = jnp.where(adj, e + 1, e).astype(jnp.float32)
    f = mf - 1.0
    y = jnp.full((LANES,), -0.145785363777915, jnp.float32)
    for c in (0.21555846203071044, -0.25229384063778415,
              0.33299270095527667, -0.49997879685373253):
        y = y * f + c
    return f + (f * f) * y + e * 0.6931471805599453


def _sqrt16(x):
    yi = 0x5F3759DF - (plsc.bitcast(x, jnp.int32) >> 1)
    y = plsc.bitcast(yi, jnp.float32)
    xh = 0.5 * x
    for _ in range(2):
        y = y * (1.5 - xh * y * y)
    return x * y


def _worker_id():
    return lax.axis_index("s") * NC + lax.axis_index("c")


def _issue(hbm_refs, bufs, sem, wid, img, quarter, c):
    row0 = quarter * (PPW // 512) + c * ROWS
    descs = []
    emb_h = hbm_refs[0]
    ebuf = bufs[0]
    for d in range(D):
        descs.append(pltpu.async_copy(
            emb_h.at[img, d, pl.ds(row0, ROWS), :], ebuf.at[d], sem))
    for h, b in zip(hbm_refs[1:], bufs[1:]):
        descs.append(pltpu.async_copy(h.at[img, pl.ds(row0, ROWS), :], b, sem))
    return descs


def _pass1_body(emb_h, inst_h, kn_h, tm_h, out_h,
                ebuf0, ibuf0, kbuf0, tbuf0, ebuf1, ibuf1, kbuf1, tbuf1,
                acc, sem):
    wid = _worker_id()
    img = wid // 4
    quarter = wid - img * 4
    zeros = jnp.zeros((LANES,), jnp.float32)
    for j in range(6 * L * LANES // LANES):
        acc[pl.ds(j * LANES, LANES)] = zeros
    lane = lax.iota(jnp.int32, LANES)
    ones = jnp.ones((LANES,), jnp.float32)
    hbm = (emb_h, inst_h, kn_h, tm_h)
    slots = ((ebuf0, ibuf0, kbuf0, tbuf0), (ebuf1, ibuf1, kbuf1, tbuf1))
    pend = _issue(hbm, slots[0], sem, wid, img, quarter, 0)
    for c in range(NCHUNK):
        ebuf, ibuf, kbuf, tbuf = slots[c % 2]
        for dsc in pend:
            dsc.wait()
        if c + 1 < NCHUNK:
            pend = _issue(hbm, slots[(c + 1) % 2], sem, wid, img, quarter, c + 1)

        @plsc.parallel_loop(0, CH // LANES, unroll=UNROLL)
        def _(i):
            r = i >> 5
            cc = (i & 31) * LANES
            iv = ibuf[r, pl.ds(cc, LANES)]
            kv = kbuf[r, pl.ds(cc, LANES)]
            tv = tbuf[r, pl.ds(cc, LANES)]
            ie = jnp.where(tv > 0.5, iv, 0)
            ik = jnp.where(kv > 0.5, ie, 0)
            idxk = ik * LANES + lane
            for d in range(D):
                ev = ebuf[d, r, pl.ds(cc, LANES)]
                plsc.addupdate_scatter(acc, [idxk + d * 256], ev)
            plsc.addupdate_scatter(acc, [idxk + 4 * 256], ones)
            plsc.addupdate_scatter(acc, [ie * LANES + lane + 5 * 256], ones)
    pltpu.sync_copy(acc, out_h.at[pl.ds(wid * 1536, 1536)])


def _pass2_body(emb_h, inst_h, tm_h, sums_h, out_h,
                ebuf0, ibuf0, tbuf0, ebuf1, ibuf1, tbuf1,
                sbuf, m0, m1, m2, m3, vacc, sem):
    wid = _worker_id()
    img = wid // 4
    quarter = wid - img * 4
    lane = lax.iota(jnp.int32, LANES)
    hbm = (emb_h, inst_h, tm_h)
    slots = ((ebuf0, ibuf0, tbuf0), (ebuf1, ibuf1, tbuf1))
    pend = _issue(hbm, slots[0], sem, wid, img, quarter, 0)
    pltpu.sync_copy(sums_h.at[pl.ds(img * 96, 96)], sbuf)
    cmax = jnp.maximum(sbuf[pl.ds(4 * LANES, LANES)], 1.0)
    mrefs = (m0, m1, m2, m3)
    for d in range(D):
        md = sbuf[pl.ds(d * LANES, LANES)] / cmax
        md = jnp.where(lane == 0, 0.0, md)
        mrefs[d][...] = md
    zeros = jnp.zeros((LANES,), jnp.float32)
    for j in range(L):
        vacc[pl.ds(j * LANES, LANES)] = zeros
    for c in range(NCHUNK):
        ebuf, ibuf, tbuf = slots[c % 2]
        for dsc in pend:
            dsc.wait()
        if c + 1 < NCHUNK:
            pend = _issue(hbm, slots[(c + 1) % 2], sem, wid, img, quarter, c + 1)

        @plsc.parallel_loop(0, CH // LANES, unroll=UNROLL)
        def _(i):
            r = i >> 5
            cc = (i & 31) * LANES
            iv = ibuf[r, pl.ds(cc, LANES)]
            tv = tbuf[r, pl.ds(cc, LANES)]
            ie = jnp.where(tv > 0.5, iv, 0)
            d0 = ebuf[0, r, pl.ds(cc, LANES)] - plsc.load_gather(m0, [ie])
            d1 = ebuf[1, r, pl.ds(cc, LANES)] - plsc.load_gather(m1, [ie])
            d2 = ebuf[2, r, pl.ds(cc, LANES)] - plsc.load_gather(m2, [ie])
            d3 = ebuf[3, r, pl.ds(cc, LANES)] - plsc.load_gather(m3, [ie])
            q = d0 * d0 + d1 * d1 + d2 * d2 + d3 * d3 + 1e-12
            t = jnp.maximum(_sqrt16(q) - 0.5, 0.0)
            v = _log16(t * t + 1.0)
            plsc.addupdate_scatter(vacc, [ie * LANES + lane], v)
    pltpu.sync_copy(vacc, out_h.at[pl.ds(wid * 256, 256)])


_pass1 = pl.kernel(
    _pass1_body,
    out_type=jax.ShapeDtypeStruct((NW * 6 * 256,), jnp.float32),
    mesh=_MESH,
    compiler_params=_SC_PARAMS,
    scratch_types=[
        pltpu.VMEM((D, ROWS, 512), jnp.float32),
        pltpu.VMEM((ROWS, 512), jnp.int32),
        pltpu.VMEM((ROWS, 512), jnp.float32),
        pltpu.VMEM((ROWS, 512), jnp.float32),
        pltpu.VMEM((D, ROWS, 512), jnp.float32),
        pltpu.VMEM((ROWS, 512), jnp.int32),
        pltpu.VMEM((ROWS, 512), jnp.float32),
        pltpu.VMEM((ROWS, 512), jnp.float32),
        pltpu.VMEM((6 * 256,), jnp.float32),
        pltpu.SemaphoreType.DMA,
    ],
)

_pass2 = pl.kernel(
    _pass2_body,
    out_type=jax.ShapeDtypeStruct((NW * 256,), jnp.float32),
    mesh=_MESH,
    compiler_params=_SC_PARAMS,
    scratch_types=[
        pltpu.VMEM((D, ROWS, 512), jnp.float32),
        pltpu.VMEM((ROWS, 512), jnp.int32),
        pltpu.VMEM((ROWS, 512), jnp.float32),
        pltpu.VMEM((D, ROWS, 512), jnp.float32),
        pltpu.VMEM((ROWS, 512), jnp.int32),
        pltpu.VMEM((ROWS, 512), jnp.float32),
        pltpu.VMEM((96,), jnp.float32),
        pltpu.VMEM((LANES,), jnp.float32),
        pltpu.VMEM((LANES,), jnp.float32),
        pltpu.VMEM((LANES,), jnp.float32),
        pltpu.VMEM((LANES,), jnp.float32),
        pltpu.VMEM((256,), jnp.float32),
        pltpu.SemaphoreType.DMA,
    ],
)


def _final_body(s_ref, v_ref, o_ref):
    s = s_ref[...]
    vs = v_ref[...]
    r = lax.broadcasted_iota(jnp.int32, (L, L), 0)
    c = lax.broadcasted_iota(jnp.int32, (L, L), 1)
    eyef = (r == c).astype(jnp.float32)
    excl = (r == c) | (r == 0) | (c == 0)
    lane2 = lax.broadcasted_iota(jnp.int32, (1, L), 1)
    total = jnp.float32(0.0)
    for b in range(B):
        cnt = jnp.maximum(s[b * 6 + 4:b * 6 + 5, :], 1.0)
        ci = jnp.maximum(s[b * 6 + 5:b * 6 + 6, :], 1.0)
        mean = s[b * 6:b * 6 + 4, :] / cnt
        mean = jnp.where(lane2 == 0, 0.0, mean)
        la = vs[b:b + 1, :] / ci
        l_agg = jnp.sum(jnp.where(lane2 == 0, 0.0, la)) / 15.0
        cross = lax.dot_general(mean, mean, (((0,), (0,)), ((), ())),
                                preferred_element_type=jnp.float32)
        m2row = jnp.sum(cross * eyef, axis=0, keepdims=True)
        m2col = jnp.sum(cross * eyef, axis=1, keepdims=True)
        pd2 = jnp.maximum(m2col + m2row - 2.0 * cross, 0.0)
        pd = jnp.sqrt(pd2 + 1e-12)
        dval = jnp.log(jnp.maximum(3.0 - pd, 0.0) ** 2 + 1.0)
        l_dis = jnp.sum(jnp.where(excl, 0.0, dval)) / 210.0
        l_reg = jnp.sum(jnp.log(jnp.sqrt(m2row + 1e-12) + 1.0)) / 16.0 * 0.001
        total = total + l_agg + l_dis + l_reg
    o_ref[0, 0] = total / B


_final = pl.pallas_call(
    _final_body,
    out_shape=jax.ShapeDtypeStruct((1, 1), jnp.float32),
    out_specs=pl.BlockSpec(memory_space=pltpu.SMEM),
)


@jax.jit
def kernel(emb, instance, kernel, training_mask):
    inst = instance.astype(jnp.int32)
    p1 = _pass1(emb, inst, kernel, training_mask)
    sums6 = p1.reshape(B, 4, 6, L, LANES).sum(axis=(1, 4))
    p2 = _pass2(emb, inst, training_mask, sums6.reshape(-1))
    valsum = p2.reshape(B, 4, L, LANES).sum(axis=(1, 3))
    out = _final(sums6.reshape(B * 6, L), valsum)
    return out[0, 0]
